# baseline (speedup 1.0000x reference)
import functools

import jax
import jax.numpy as jnp
from jax import lax
from jax.experimental import pallas as pl
from jax.experimental.pallas import tpu as pltpu
from jax.experimental.pallas import tpu_sc as plsc

N, D, K = 4096, 64, 512
BLK = 512
NW = 32
RPW = N // NW


def _tc_dist_body(x_ref, c_ref, d2_ref, out_min_ref):
    xb = x_ref[...]
    c = c_ref[...]
    g = lax.dot_general(
        xb, c, (((1,), (1,)), ((), ())),
        preferred_element_type=jnp.float32,
        precision=lax.Precision.HIGHEST,
    )
    xn = jnp.sum(xb * xb, axis=1)
    cn = jnp.sum(c * c, axis=1)
    d2 = jnp.maximum(xn[:, None] + cn[None, :] - 2.0 * g, 0.0)
    d2_ref[...] = d2
    out_min_ref[...] = jnp.sqrt(jnp.min(d2, axis=1))


def _tc_dist(x, centers):
    return pl.pallas_call(
        _tc_dist_body,
        grid=(N // BLK,),
        in_specs=[
            pl.BlockSpec((BLK, D), lambda i: (i, 0)),
            pl.BlockSpec((K, D), lambda i: (0, 0)),
        ],
        out_specs=[
            pl.BlockSpec((BLK, K), lambda i: (i, 0)),
            pl.BlockSpec((BLK,), lambda i: (i,)),
        ],
        out_shape=[
            jax.ShapeDtypeStruct((N, K), jnp.float32),
            jax.ShapeDtypeStruct((N,), jnp.float32),
        ],
        compiler_params=pltpu.CompilerParams(
            dimension_semantics=("arbitrary",),
        ),
    )(x, centers)


def _sc_colmin_body(d2_hbm, bval_hbm, bidx_hbm, d2_v, bv_v, bi_v):
    cid = lax.axis_index("c")
    sid = lax.axis_index("s")
    wid = sid * 2 + cid
    base = wid * RPW
    pltpu.sync_copy(d2_hbm.at[pl.ds(base, RPW), :], d2_v)

    def j_loop(j, _):
        col0 = j * 16

        def p_loop(p, carry):
            bv, bi = carry
            v = d2_v[p, pl.ds(col0, 16)]
            idx = jnp.full((16,), base + p, dtype=jnp.int32)
            m = v < bv
            return jnp.where(m, v, bv), jnp.where(m, idx, bi)

        bv0 = jnp.full((16,), jnp.inf, dtype=jnp.float32)
        bi0 = jnp.full((16,), N, dtype=jnp.int32)
        bv, bi = lax.fori_loop(0, RPW, p_loop, (bv0, bi0))
        bv_v[pl.ds(col0, 16)] = bv
        bi_v[pl.ds(col0, 16)] = bi
        return 0

    lax.fori_loop(0, K // 16, j_loop, 0)
    pltpu.sync_copy(bv_v, bval_hbm.at[wid])
    pltpu.sync_copy(bi_v, bidx_hbm.at[wid])


@functools.partial(
    pl.kernel,
    out_type=[
        jax.ShapeDtypeStruct((NW, K), jnp.float32),
        jax.ShapeDtypeStruct((NW, K), jnp.int32),
    ],
    mesh=plsc.VectorSubcoreMesh(core_axis_name="c", subcore_axis_name="s"),
    scratch_types=[
        pltpu.VMEM((RPW, K), jnp.float32),
        pltpu.VMEM((K,), jnp.float32),
        pltpu.VMEM((K,), jnp.int32),
    ],
)
def _sc_colmin(d2_hbm, bval_hbm, bidx_hbm, d2_v, bv_v, bi_v):
    _sc_colmin_body(d2_hbm, bval_hbm, bidx_hbm, d2_v, bv_v, bi_v)


def _tc_merge_body(bval_ref, bidx_ref, out_idx_ref):
    bval = bval_ref[...]
    bidx = bidx_ref[...]
    minv = jnp.min(bval, axis=0)
    out_idx_ref[...] = jnp.min(
        jnp.where(bval == minv[None, :], bidx, jnp.int32(N)), axis=0
    )


def _tc_merge(bval, bidx):
    return pl.pallas_call(
        _tc_merge_body,
        out_shape=jax.ShapeDtypeStruct((K,), jnp.int32),
    )(bval, bidx)


@jax.jit
def kernel(x, centers):
    d2, out_min = _tc_dist(x, centers)
    bval, bidx = _sc_colmin(d2)
    out_idx = _tc_merge(bval, bidx)
    return out_idx, out_min, centers


# speedup vs baseline: 1.2394x; 1.2394x over previous
import functools

import jax
import jax.numpy as jnp
from jax import lax
from jax.experimental import pallas as pl
from jax.experimental.pallas import tpu as pltpu
from jax.experimental.pallas import tpu_sc as plsc

N, D, K = 4096, 64, 512
BLK = 512
NW = 32
RPW = N // NW


def _tc_dist_body(x_ref, c_ref, d2_ref, out_min_ref):
    xb = x_ref[...]
    c = c_ref[...]
    g = lax.dot_general(
        xb, c, (((1,), (1,)), ((), ())),
        preferred_element_type=jnp.float32,
        precision=lax.Precision.HIGHEST,
    )
    xn = jnp.sum(xb * xb, axis=1)
    cn = jnp.sum(c * c, axis=1)
    d2 = jnp.maximum(xn[:, None] + cn[None, :] - 2.0 * g, 0.0)
    d2_ref[...] = d2
    out_min_ref[...] = jnp.sqrt(jnp.min(d2, axis=1))


def _tc_dist(x, centers):
    return pl.pallas_call(
        _tc_dist_body,
        grid=(N // BLK,),
        in_specs=[
            pl.BlockSpec((BLK, D), lambda i: (i, 0)),
            pl.BlockSpec((K, D), lambda i: (0, 0)),
        ],
        out_specs=[
            pl.BlockSpec((BLK, K), lambda i: (i, 0)),
            pl.BlockSpec((BLK,), lambda i: (i,)),
        ],
        out_shape=[
            jax.ShapeDtypeStruct((N, K), jnp.float32),
            jax.ShapeDtypeStruct((N,), jnp.float32),
        ],
        compiler_params=pltpu.CompilerParams(
            dimension_semantics=("arbitrary",),
        ),
    )(x, centers)


CH = 32
NCH = RPW // CH


def _sc_colmin_body(d2_hbm, bval_hbm, bidx_hbm, buf_v, bv_v, bi_v, sem_a, sem_b):
    cid = lax.axis_index("c")
    sid = lax.axis_index("s")
    wid = sid * 2 + cid
    base = wid * RPW

    inf16 = jnp.full((16,), jnp.inf, dtype=jnp.float32)
    n16 = jnp.full((16,), N, dtype=jnp.int32)

    def init_loop(j, _):
        bv_v[pl.ds(j * 16, 16)] = inf16
        bi_v[pl.ds(j * 16, 16)] = n16
        return 0

    lax.fori_loop(0, K // 16, init_loop, 0)

    sems = (sem_a, sem_b)
    copies = [None, None]
    copies[0] = pltpu.async_copy(
        d2_hbm.at[pl.ds(base, CH), :], buf_v.at[0], sems[0]
    )
    for ch in range(NCH):
        cur = ch % 2
        nxt = (ch + 1) % 2
        if ch + 1 < NCH:
            copies[nxt] = pltpu.async_copy(
                d2_hbm.at[pl.ds(base + (ch + 1) * CH, CH), :],
                buf_v.at[nxt],
                sems[nxt],
            )
        copies[cur].wait()
        p0glob = base + ch * CH

        def jg_loop(jg, _, cur=cur, p0glob=p0glob):
            col = jg * 64
            bvs = tuple(bv_v[pl.ds(col + c * 16, 16)] for c in range(4))
            bis = tuple(bi_v[pl.ds(col + c * 16, 16)] for c in range(4))

            def p_loop(p, carry):
                cbv, cbi = carry
                idx = jnp.full((16,), p0glob + p, dtype=jnp.int32)
                nbv, nbi = [], []
                for c in range(4):
                    v = buf_v[cur, p, pl.ds(col + c * 16, 16)]
                    m = v < cbv[c]
                    nbv.append(jnp.where(m, v, cbv[c]))
                    nbi.append(jnp.where(m, idx, cbi[c]))
                return tuple(nbv), tuple(nbi)

            bvs, bis = lax.fori_loop(0, CH, p_loop, (bvs, bis))
            for c in range(4):
                bv_v[pl.ds(col + c * 16, 16)] = bvs[c]
                bi_v[pl.ds(col + c * 16, 16)] = bis[c]
            return 0

        lax.fori_loop(0, K // 64, jg_loop, 0)

    pltpu.sync_copy(bv_v, bval_hbm.at[wid])
    pltpu.sync_copy(bi_v, bidx_hbm.at[wid])


@functools.partial(
    pl.kernel,
    out_type=[
        jax.ShapeDtypeStruct((NW, K), jnp.float32),
        jax.ShapeDtypeStruct((NW, K), jnp.int32),
    ],
    mesh=plsc.VectorSubcoreMesh(core_axis_name="c", subcore_axis_name="s"),
    scratch_types=[
        pltpu.VMEM((2, CH, K), jnp.float32),
        pltpu.VMEM((K,), jnp.float32),
        pltpu.VMEM((K,), jnp.int32),
        pltpu.SemaphoreType.DMA,
        pltpu.SemaphoreType.DMA,
    ],
)
def _sc_colmin(d2_hbm, bval_hbm, bidx_hbm, buf_v, bv_v, bi_v, sem_a, sem_b):
    _sc_colmin_body(d2_hbm, bval_hbm, bidx_hbm, buf_v, bv_v, bi_v, sem_a, sem_b)


def _tc_merge_body(bval_ref, bidx_ref, out_idx_ref):
    bval = bval_ref[...]
    bidx = bidx_ref[...]
    minv = jnp.min(bval, axis=0)
    out_idx_ref[...] = jnp.min(
        jnp.where(bval == minv[None, :], bidx, jnp.int32(N)), axis=0
    )


def _tc_merge(bval, bidx):
    return pl.pallas_call(
        _tc_merge_body,
        out_shape=jax.ShapeDtypeStruct((K,), jnp.int32),
    )(bval, bidx)


@jax.jit
def kernel(x, centers):
    d2, out_min = _tc_dist(x, centers)
    bval, bidx = _sc_colmin(d2)
    out_idx = _tc_merge(bval, bidx)
    return out_idx, out_min, centers
